# Initial kernel scaffold; baseline (speedup 1.0000x reference)
import functools

import jax
import jax.numpy as jnp
from jax import lax
from jax.experimental import pallas as pl
from jax.experimental.pallas import tpu as pltpu
from jax.experimental.pallas import tpu_sc as plsc

N = 10000
E = 320000
D = 128

NC = 2
NS = 16
NW = NC * NS
EPW = E // NW
K = 80
NB = EPW // K
RPT = N // NS


def _leaky(v):
    return jnp.maximum(v, 0.01 * v)


def _prep_body(x_ref, pos_ref, w1t_ref, b1_ref, w2t_ref, b2_ref,
               wfrt_ref, wfxt_ref, bf_ref, a_ref, b_ref):
    x = x_ref[...]
    pos = pos_ref[...]
    h = _leaky(jnp.dot(x, w1t_ref[...], preferred_element_type=jnp.float32)
               + b1_ref[...])
    delta = jnp.tanh(
        jnp.dot(h, w2t_ref[...], preferred_element_type=jnp.float32)
        + b2_ref[...])
    wfrt = wfrt_ref[...]
    a_ref[...] = (jnp.dot(x, wfxt_ref[...], preferred_element_type=jnp.float32)
                  + jnp.dot(pos, wfrt, preferred_element_type=jnp.float32))
    b_ref[...] = (jnp.dot(delta - pos, wfrt,
                          preferred_element_type=jnp.float32)
                  + bf_ref[...])


def _post_body(p_ref, wg1t_ref, bg1_ref, wg2t_ref, bg2_ref, out_ref):
    agg = p_ref[0] + p_ref[1]
    t = _leaky(jnp.dot(agg, wg1t_ref[...], preferred_element_type=jnp.float32)
               + bg1_ref[...])
    out_ref[...] = (jnp.dot(t, wg2t_ref[...],
                            preferred_element_type=jnp.float32)
                    + bg2_ref[...])


_mesh = plsc.VectorSubcoreMesh(core_axis_name="c", subcore_axis_name="s")


@functools.partial(
    pl.kernel,
    mesh=_mesh,
    out_type=jax.ShapeDtypeStruct((NC, N, D), jnp.float32),
    scratch_types=[
        pltpu.VMEM((K,), jnp.int32),
        pltpu.VMEM((K,), jnp.int32),
        pltpu.VMEM((K, D), jnp.float32),
        pltpu.VMEM((K, D), jnp.float32),
        pltpu.VMEM_SHARED((N, D), jnp.float32),
        pltpu.SemaphoreType.DMA,
        pltpu.SemaphoreType.DMA,
    ],
)
def _edge_kernel(a_hbm, b_hbm, src_hbm, dst_hbm, zero_hbm, out_hbm,
                 src_v, dst_v, a_rows, b_rows, agg_sh, sem_a, sem_b):
    c = lax.axis_index("c")
    s = lax.axis_index("s")
    wid = s * NC + c

    pltpu.sync_copy(zero_hbm, agg_sh.at[pl.ds(s * RPT, RPT)])
    plsc.subcore_barrier()

    def block(t, carry):
        off = wid * EPW + t * K
        pltpu.sync_copy(src_hbm.at[pl.ds(off, K)], src_v)
        pltpu.sync_copy(dst_hbm.at[pl.ds(off, K)], dst_v)
        ga = pltpu.async_copy(a_hbm.at[src_v], a_rows, sem_a)
        gb = pltpu.async_copy(b_hbm.at[dst_v], b_rows, sem_b)
        ga.wait()
        gb.wait()

        def row(r, rc):
            for j in range(D // 16):
                sl = pl.ds(j * 16, 16)
                m = a_rows[r, sl] + b_rows[r, sl]
                a_rows[r, sl] = jnp.maximum(m, m * 0.01)
            return rc

        lax.fori_loop(0, K, row, 0)
        pltpu.sync_copy(a_rows, agg_sh.at[dst_v], add=True)
        return carry

    lax.fori_loop(0, NB, block, 0)

    plsc.subcore_barrier()
    pltpu.sync_copy(agg_sh.at[pl.ds(s * RPT, RPT)],
                    out_hbm.at[c, pl.ds(s * RPT, RPT)])


def kernel(x, pos, edge_index, W1h, b1h, W2h, b2h, Wf, bf, Wg1, bg1, Wg2, bg2):
    x = x.astype(jnp.float32)
    pos = pos.astype(jnp.float32)
    src = edge_index[0].astype(jnp.int32)
    dst = edge_index[1].astype(jnp.int32)

    a, b = pl.pallas_call(
        _prep_body,
        out_shape=[jax.ShapeDtypeStruct((N, D), jnp.float32),
                   jax.ShapeDtypeStruct((N, D), jnp.float32)],
    )(x, pos, W1h.T, b1h.reshape(1, D), W2h.T, b2h.reshape(1, 3),
      Wf[:, :3].T, Wf[:, 3:].T, bf.reshape(1, D))

    zeros = jnp.zeros((RPT, D), jnp.float32)
    partials = _edge_kernel(a, b, src, dst, zeros)

    out = pl.pallas_call(
        _post_body,
        out_shape=jax.ShapeDtypeStruct((N, D), jnp.float32),
    )(partials, Wg1.T, bg1.reshape(1, D), Wg2.T, bg2.reshape(1, D))
    return out


# speedup vs baseline: 8.1750x; 8.1750x over previous
import functools

import jax
import jax.numpy as jnp
from jax import lax
from jax.experimental import pallas as pl
from jax.experimental.pallas import tpu as pltpu
from jax.experimental.pallas import tpu_sc as plsc

N = 10000
E = 320000
D = 128

NC = 2
NS = 16
NW = NC * NS
EPW = E // NW
K = 80
NB = EPW // K
NPAD = 10240
RPT = NPAD // NS


def _leaky(v):
    return jnp.maximum(v, 0.01 * v)


def _prep_body(x_ref, pos_ref, w1t_ref, b1_ref, w2t_ref, b2_ref,
               wfrt_ref, wfxt_ref, bf_ref, a_ref, b_ref):
    x = x_ref[...]
    pos = pos_ref[...]
    h = _leaky(jnp.dot(x, w1t_ref[...], preferred_element_type=jnp.float32)
               + b1_ref[...])
    delta = jnp.tanh(
        jnp.dot(h, w2t_ref[...], preferred_element_type=jnp.float32)
        + b2_ref[...])
    wfrt = wfrt_ref[...]
    a_ref[...] = (jnp.dot(x, wfxt_ref[...], preferred_element_type=jnp.float32)
                  + jnp.dot(pos, wfrt, preferred_element_type=jnp.float32))
    b_ref[...] = (jnp.dot(delta - pos, wfrt,
                          preferred_element_type=jnp.float32)
                  + bf_ref[...])


def _post_body(p_ref, wg1t_ref, bg1_ref, wg2t_ref, bg2_ref, out_ref):
    agg = p_ref[0, :N] + p_ref[1, :N]
    t = _leaky(jnp.dot(agg, wg1t_ref[...], preferred_element_type=jnp.float32)
               + bg1_ref[...])
    out_ref[...] = (jnp.dot(t, wg2t_ref[...],
                            preferred_element_type=jnp.float32)
                    + bg2_ref[...])


_mesh = plsc.VectorSubcoreMesh(core_axis_name="c", subcore_axis_name="s")


@functools.partial(
    pl.kernel,
    mesh=_mesh,
    out_type=jax.ShapeDtypeStruct((NC, NPAD, D), jnp.float32),
    scratch_types=[
        pltpu.VMEM((K,), jnp.int32),
        pltpu.VMEM((K,), jnp.int32),
        pltpu.VMEM((K, D), jnp.float32),
        pltpu.VMEM((K, D), jnp.float32),
        pltpu.VMEM_SHARED((NPAD, D), jnp.float32),
        pltpu.SemaphoreType.DMA,
        pltpu.SemaphoreType.DMA,
    ],
)
def _edge_kernel(a_hbm, b_hbm, src_hbm, dst_hbm, zero_hbm, out_hbm,
                 src_v, dst_v, a_rows, b_rows, agg_sh, sem_a, sem_b):
    c = lax.axis_index("c")
    s = lax.axis_index("s")
    wid = s * NC + c

    pltpu.sync_copy(zero_hbm, agg_sh.at[pl.ds(s * RPT, RPT)])
    plsc.subcore_barrier()

    def block(t, carry):
        off = wid * EPW + t * K
        pltpu.sync_copy(src_hbm.at[pl.ds(off, K)], src_v)
        pltpu.sync_copy(dst_hbm.at[pl.ds(off, K)], dst_v)
        ga = pltpu.async_copy(a_hbm.at[src_v], a_rows, sem_a)
        gb = pltpu.async_copy(b_hbm.at[dst_v], b_rows, sem_b)
        ga.wait()
        gb.wait()

        def row(r, rc):
            for j in range(D // 16):
                sl = pl.ds(j * 16, 16)
                m = a_rows[r, sl] + b_rows[r, sl]
                a_rows[r, sl] = jnp.maximum(m, m * 0.01)
            return rc

        lax.fori_loop(0, K, row, 0)
        pltpu.sync_copy(a_rows, agg_sh.at[dst_v], add=True)
        return carry

    lax.fori_loop(0, NB, block, 0)

    plsc.subcore_barrier()
    pltpu.sync_copy(agg_sh.at[pl.ds(s * RPT, RPT)],
                    out_hbm.at[c, pl.ds(s * RPT, RPT)])


def kernel(x, pos, edge_index, W1h, b1h, W2h, b2h, Wf, bf, Wg1, bg1, Wg2, bg2):
    x = x.astype(jnp.float32)
    pos = pos.astype(jnp.float32)
    src = edge_index[0].astype(jnp.int32)
    dst = edge_index[1].astype(jnp.int32)

    a, b = pl.pallas_call(
        _prep_body,
        out_shape=[jax.ShapeDtypeStruct((N, D), jnp.float32),
                   jax.ShapeDtypeStruct((N, D), jnp.float32)],
    )(x, pos, W1h.T, b1h.reshape(1, D), W2h.T, b2h.reshape(1, 3),
      Wf[:, :3].T, Wf[:, 3:].T, bf.reshape(1, D))

    zeros = jnp.zeros((RPT, D), jnp.float32)
    partials = _edge_kernel(a, b, src, dst, zeros)

    out = pl.pallas_call(
        _post_body,
        out_shape=jax.ShapeDtypeStruct((N, D), jnp.float32),
    )(partials, Wg1.T, bg1.reshape(1, D), Wg2.T, bg2.reshape(1, D))
    return out


# speedup vs baseline: 13.7470x; 1.6816x over previous
import functools

import jax
import jax.numpy as jnp
from jax import lax
from jax.experimental import pallas as pl
from jax.experimental.pallas import tpu as pltpu
from jax.experimental.pallas import tpu_sc as plsc

N = 10000
E = 320000
D = 128

NC = 2
NS = 16
NW = NC * NS
EPW = E // NW
K = 40
NV = 5
NB = EPW // (K * NV)
NPAD = 10240
RPT = NPAD // NS


def _leaky(v):
    return jnp.maximum(v, 0.01 * v)


def _prep_body(x_ref, pos_ref, w1t_ref, b1_ref, w2t_ref, b2_ref,
               wfrt_ref, wfxt_ref, bf_ref, a_ref, b_ref):
    x = x_ref[...]
    pos = pos_ref[...]
    h = _leaky(jnp.dot(x, w1t_ref[...], preferred_element_type=jnp.float32)
               + b1_ref[...])
    delta = jnp.tanh(
        jnp.dot(h, w2t_ref[...], preferred_element_type=jnp.float32)
        + b2_ref[...])
    wfrt = wfrt_ref[...]
    a_ref[...] = (jnp.dot(x, wfxt_ref[...], preferred_element_type=jnp.float32)
                  + jnp.dot(pos, wfrt, preferred_element_type=jnp.float32))
    b_ref[...] = (jnp.dot(delta - pos, wfrt,
                          preferred_element_type=jnp.float32)
                  + bf_ref[...])


def _post_body(p_ref, wg1t_ref, bg1_ref, wg2t_ref, bg2_ref, out_ref):
    agg = p_ref[0, :N] + p_ref[1, :N]
    t = _leaky(jnp.dot(agg, wg1t_ref[...], preferred_element_type=jnp.float32)
               + bg1_ref[...])
    out_ref[...] = (jnp.dot(t, wg2t_ref[...],
                            preferred_element_type=jnp.float32)
                    + bg2_ref[...])


_mesh = plsc.VectorSubcoreMesh(core_axis_name="c", subcore_axis_name="s")


@functools.partial(
    pl.kernel,
    mesh=_mesh,
    out_type=jax.ShapeDtypeStruct((NC, NPAD, D), jnp.float32),
    scratch_types=[
        pltpu.VMEM((NB, 1, K), jnp.int32),
        pltpu.VMEM((NB, 1, K), jnp.int32),
        pltpu.VMEM((K, D), jnp.float32),
        pltpu.VMEM((K, D), jnp.float32),
        pltpu.VMEM((K, D), jnp.float32),
        pltpu.VMEM((K, D), jnp.float32),
        pltpu.VMEM((K, D), jnp.float32),
        pltpu.VMEM((K, D), jnp.float32),
        pltpu.VMEM_SHARED((NPAD, D), jnp.float32),
        pltpu.SemaphoreType.DMA,
        pltpu.SemaphoreType.DMA,
        pltpu.SemaphoreType.DMA,
        pltpu.SemaphoreType.DMA,
        pltpu.SemaphoreType.DMA,
        pltpu.SemaphoreType.DMA,
    ],
)
def _edge_kernel(a_hbm, b_hbm, src_hbm, dst_hbm, zero_hbm, out_hbm,
                 src_v, dst_v, a0, a1, b0, b1, e0, e1,
                 agg_sh, sa0, sa1, sb0, sb1, ss0, ss1):
    c = lax.axis_index("c")
    s = lax.axis_index("s")
    wid = s * NC + c

    pltpu.sync_copy(zero_hbm, agg_sh.at[pl.ds(s * RPT, RPT)])
    plsc.subcore_barrier()

    slots = ((a0, b0, e0, sa0, sb0, ss0),
             (a1, b1, e1, sa1, sb1, ss1))

    def issue_gathers(t, slot):
        a_r, b_r, _, sa, sb, _ = slots[slot]
        pltpu.async_copy(a_hbm.at[src_v.at[t, 0]], a_r, sa)
        pltpu.async_copy(b_hbm.at[dst_v.at[t, 0]], b_r, sb)

    def wait_gathers(slot):
        a_r, b_r, _, sa, sb, _ = slots[slot]
        pltpu.make_async_copy(a_hbm.at[src_v.at[0, 0]], a_r, sa).wait()
        pltpu.make_async_copy(b_hbm.at[dst_v.at[0, 0]], b_r, sb).wait()

    def issue_scatter(t, slot):
        _, _, e_r, _, _, ss = slots[slot]
        pltpu.async_copy(e_r, agg_sh.at[dst_v.at[t, 0]], ss, add=True)

    def wait_scatter(slot):
        _, _, e_r, _, _, ss = slots[slot]
        pltpu.make_async_copy(e_r, agg_sh.at[dst_v.at[0, 0]], ss).wait()

    def compute_block(slot):
        a_r, b_r, e_r, *_ = slots[slot]

        def row(r, rc):
            for u in range(2):
                for j in range(D // 16):
                    sl = pl.ds(j * 16, 16)
                    m = a_r[2 * r + u, sl] + b_r[2 * r + u, sl]
                    e_r[2 * r + u, sl] = jnp.maximum(m, m * 0.01)
            return rc

        lax.fori_loop(0, K // 2, row, 0)

    for v in range(NV):
        pltpu.sync_copy(src_hbm.at[wid, v], src_v)
        pltpu.sync_copy(dst_hbm.at[wid, v], dst_v)

        issue_gathers(0, 0)
        issue_gathers(1, 1)

        def outer(t0, carry):
            for bslot in range(2):
                t = 2 * t0 + bslot
                wait_gathers(bslot)

                @pl.when(t0 >= 1)
                def _():
                    wait_scatter(bslot)

                compute_block(bslot)
                issue_scatter(t, bslot)
                issue_gathers(jnp.minimum(t + 2, NB - 1), bslot)
            return carry

        lax.fori_loop(0, NB // 2, outer, 0)

        wait_gathers(0)
        wait_gathers(1)
        wait_scatter(0)
        wait_scatter(1)

    plsc.subcore_barrier()
    pltpu.sync_copy(agg_sh.at[pl.ds(s * RPT, RPT)],
                    out_hbm.at[c, pl.ds(s * RPT, RPT)])


def kernel(x, pos, edge_index, W1h, b1h, W2h, b2h, Wf, bf, Wg1, bg1, Wg2, bg2):
    x = x.astype(jnp.float32)
    pos = pos.astype(jnp.float32)
    src = edge_index[0].astype(jnp.int32).reshape(NW, NV, NB, 1, K)
    dst = edge_index[1].astype(jnp.int32).reshape(NW, NV, NB, 1, K)

    a, b = pl.pallas_call(
        _prep_body,
        out_shape=[jax.ShapeDtypeStruct((N, D), jnp.float32),
                   jax.ShapeDtypeStruct((N, D), jnp.float32)],
    )(x, pos, W1h.T, b1h.reshape(1, D), W2h.T, b2h.reshape(1, 3),
      Wf[:, :3].T, Wf[:, 3:].T, bf.reshape(1, D))

    zeros = jnp.zeros((RPT, D), jnp.float32)
    partials = _edge_kernel(a, b, src, dst, zeros)

    out = pl.pallas_call(
        _post_body,
        out_shape=jax.ShapeDtypeStruct((N, D), jnp.float32),
    )(partials, Wg1.T, bg1.reshape(1, D), Wg2.T, bg2.reshape(1, D))
    return out
